# Initial kernel scaffold; baseline (speedup 1.0000x reference)
import jax
import jax.numpy as jnp
from jax.experimental import pallas as pl


def kernel(x, edge_index, edge_attr, params):
    raise NotImplementedError("write your pallas kernel here")


# speedup vs baseline: 2.2564x; 2.2564x over previous
import functools

import jax
import jax.numpy as jnp
import numpy as np
from jax import lax
from jax.experimental import pallas as pl
from jax.experimental.pallas import tpu as pltpu
from jax.experimental.pallas import tpu_sc as plsc

N = 10000
E = 320000
H = 96
NC = 2
NS = 16
NW = NC * NS
EW = E // NW
CH = 80
NCHUNK = EW // CH
NROW = N // NS

TE = 2560
GE = E // TE
TN = 2000
GN = N // TN

_SQRT1_2 = np.float32(0.70710678118654752440)


def _gelu(x):
    return x * (0.5 * (1.0 + lax.erf(x * _SQRT1_2)))


def _ln(y, gam, bet):
    mu = jnp.mean(y, axis=-1, keepdims=True)
    yc = y - mu
    var = jnp.mean(yc * yc, axis=-1, keepdims=True)
    return yc * lax.rsqrt(var + 1e-5) * gam + bet


def _dot(a, b):
    return jnp.dot(a, b, preferred_element_type=jnp.float32)


_sc_mesh = plsc.VectorSubcoreMesh(
    core_axis_name="c", subcore_axis_name="s", num_cores=NC, num_subcores=NS)

_sc_params = pltpu.CompilerParams(use_tc_tiling_on_sc=False)


@functools.partial(
    pl.kernel,
    out_type=jax.ShapeDtypeStruct((E, H), jnp.float32),
    mesh=_sc_mesh,
    scratch_types=[
        pltpu.VMEM((CH,), jnp.int32),
        pltpu.VMEM((CH,), jnp.int32),
        pltpu.VMEM((CH, H), jnp.float32),
        pltpu.VMEM((CH, H), jnp.float32),
        pltpu.SemaphoreType.DMA,
        pltpu.SemaphoreType.DMA,
    ],
    compiler_params=_sc_params,
)
def _gather_pair(a_hbm, b_hbm, src_hbm, dst_hbm, out_hbm,
                 si_v, di_v, ra_v, rb_v, sem_a, sem_b):
    wid = lax.axis_index("s") * NC + lax.axis_index("c")
    base = wid * EW

    def chunk(i, carry):
        off = base + i * CH
        pltpu.sync_copy(src_hbm.at[pl.ds(off, CH)], si_v)
        pltpu.sync_copy(dst_hbm.at[pl.ds(off, CH)], di_v)
        cpa = pltpu.async_copy(a_hbm.at[si_v], ra_v, sem_a)
        cpb = pltpu.async_copy(b_hbm.at[di_v], rb_v, sem_b)
        cpa.wait()
        cpb.wait()

        def row(r, c2):
            for j in range(H // 16):
                sl = pl.ds(j * 16, 16)
                ra_v[r, sl] = ra_v[r, sl] + rb_v[r, sl]
            return c2

        lax.fori_loop(0, CH, row, 0)
        pltpu.sync_copy(ra_v, out_hbm.at[pl.ds(off, CH)])
        return carry

    lax.fori_loop(0, NCHUNK, chunk, 0)


@functools.partial(
    pl.kernel,
    out_type=jax.ShapeDtypeStruct((NC, N, H), jnp.float32),
    mesh=_sc_mesh,
    scratch_types=[
        pltpu.VMEM((CH,), jnp.int32),
        pltpu.VMEM((CH, H), jnp.float32),
        pltpu.VMEM((NROW, H), jnp.float32),
        pltpu.VMEM_SHARED((N, H), jnp.float32),
        pltpu.SemaphoreType.DMA,
    ],
    compiler_params=_sc_params,
)
def _scatter_sum(e_hbm, dst_hbm, out_hbm, di_v, er_v, z_v, acc_sh, sem):
    cid = lax.axis_index("c")
    sid = lax.axis_index("s")
    wid = sid * NC + cid
    base = wid * EW

    zero = jnp.zeros((16,), jnp.float32)

    def zrow(r, carry):
        for j in range(H // 16):
            z_v[r, pl.ds(j * 16, 16)] = zero
        return carry

    lax.fori_loop(0, NROW, zrow, 0)
    pltpu.sync_copy(z_v, acc_sh.at[pl.ds(sid * NROW, NROW)])
    plsc.subcore_barrier()

    def chunk(i, carry):
        off = base + i * CH
        pltpu.sync_copy(dst_hbm.at[pl.ds(off, CH)], di_v)
        pltpu.async_copy(e_hbm.at[pl.ds(off, CH)], er_v, sem).wait()
        pltpu.sync_copy(er_v, acc_sh.at[di_v], add=True)
        return carry

    lax.fori_loop(0, NCHUNK, chunk, 0)
    plsc.subcore_barrier()
    pltpu.sync_copy(acc_sh.at[pl.ds(sid * NROW, NROW)],
                    out_hbm.at[cid, pl.ds(sid * NROW, NROW)])


def _vec_spec():
    return pl.BlockSpec((1, H), lambda i: (0, 0))


def _mat_spec(d0=H, d1=H):
    return pl.BlockSpec((d0, d1), lambda i: (0, 0))


def _edge_enc_body(ea_ref, w1_ref, b1_ref, w2_ref, b2_ref, out_ref):
    t = _dot(ea_ref[...], w1_ref[...]) + b1_ref[...]
    out_ref[...] = _dot(_gelu(t), w2_ref[...]) + b2_ref[...]


_edge_enc = pl.pallas_call(
    _edge_enc_body,
    grid=(GE,),
    in_specs=[
        pl.BlockSpec((TE, 8), lambda i: (i, 0)),
        pl.BlockSpec((8, H), lambda i: (0, 0)),
        _vec_spec(),
        _mat_spec(),
        _vec_spec(),
    ],
    out_specs=pl.BlockSpec((TE, H), lambda i: (i, 0)),
    out_shape=jax.ShapeDtypeStruct((E, H), jnp.float32),
)


def _node_enc_body(x_ref, w1_ref, b1_ref, w2_ref, b2_ref,
                   w1a_ref, w1b_ref, eb1_ref,
                   h_ref, a_ref, b_ref):
    t = _dot(x_ref[...], w1_ref[...]) + b1_ref[...]
    h = _dot(_gelu(t), w2_ref[...]) + b2_ref[...]
    h_ref[...] = h
    a_ref[...] = _dot(h, w1a_ref[...]) + eb1_ref[...]
    b_ref[...] = _dot(h, w1b_ref[...])


_node_enc = pl.pallas_call(
    _node_enc_body,
    grid=(GN,),
    in_specs=[
        pl.BlockSpec((TN, 8), lambda i: (i, 0)),
        pl.BlockSpec((8, H), lambda i: (0, 0)),
        _vec_spec(),
        _mat_spec(),
        _vec_spec(),
        _mat_spec(),
        _mat_spec(),
        _vec_spec(),
    ],
    out_specs=[
        pl.BlockSpec((TN, H), lambda i: (i, 0)),
        pl.BlockSpec((TN, H), lambda i: (i, 0)),
        pl.BlockSpec((TN, H), lambda i: (i, 0)),
    ],
    out_shape=[
        jax.ShapeDtypeStruct((N, H), jnp.float32),
        jax.ShapeDtypeStruct((N, H), jnp.float32),
        jax.ShapeDtypeStruct((N, H), jnp.float32),
    ],
)


def _edge_update_body(e_ref, g_ref, w1c_ref, w2_ref, b2_ref, gam_ref, bet_ref,
                      out_ref):
    e = e_ref[...]
    t = g_ref[...] + _dot(e, w1c_ref[...])
    y = e + _dot(_gelu(t), w2_ref[...]) + b2_ref[...]
    out_ref[...] = _ln(y, gam_ref[...], bet_ref[...])


_edge_update = pl.pallas_call(
    _edge_update_body,
    grid=(GE,),
    in_specs=[
        pl.BlockSpec((TE, H), lambda i: (i, 0)),
        pl.BlockSpec((TE, H), lambda i: (i, 0)),
        _mat_spec(),
        _mat_spec(),
        _vec_spec(),
        _vec_spec(),
        _vec_spec(),
    ],
    out_specs=pl.BlockSpec((TE, H), lambda i: (i, 0)),
    out_shape=jax.ShapeDtypeStruct((E, H), jnp.float32),
)


def _node_update_body(h_ref, g0_ref, g1_ref,
                      v1h_ref, v1a_ref, vb1_ref, v2_ref, vb2_ref,
                      gam_ref, bet_ref,
                      w1a_ref, w1b_ref, eb1_ref,
                      h_out, a_out, b_out):
    h = h_ref[...]
    agg = g0_ref[...] + g1_ref[...]
    t = _dot(h, v1h_ref[...]) + _dot(agg, v1a_ref[...]) + vb1_ref[...]
    y = h + _dot(_gelu(t), v2_ref[...]) + vb2_ref[...]
    hn = _ln(y, gam_ref[...], bet_ref[...])
    h_out[...] = hn
    a_out[...] = _dot(hn, w1a_ref[...]) + eb1_ref[...]
    b_out[...] = _dot(hn, w1b_ref[...])


_node_update = pl.pallas_call(
    _node_update_body,
    grid=(GN,),
    in_specs=[
        pl.BlockSpec((TN, H), lambda i: (i, 0)),
        pl.BlockSpec((TN, H), lambda i: (i, 0)),
        pl.BlockSpec((TN, H), lambda i: (i, 0)),
        _mat_spec(), _mat_spec(), _vec_spec(), _mat_spec(), _vec_spec(),
        _vec_spec(), _vec_spec(),
        _mat_spec(), _mat_spec(), _vec_spec(),
    ],
    out_specs=[
        pl.BlockSpec((TN, H), lambda i: (i, 0)),
        pl.BlockSpec((TN, H), lambda i: (i, 0)),
        pl.BlockSpec((TN, H), lambda i: (i, 0)),
    ],
    out_shape=[
        jax.ShapeDtypeStruct((N, H), jnp.float32),
        jax.ShapeDtypeStruct((N, H), jnp.float32),
        jax.ShapeDtypeStruct((N, H), jnp.float32),
    ],
)


def _node_last_body(h_ref, g0_ref, g1_ref,
                    v1h_ref, v1a_ref, vb1_ref, v2_ref, vb2_ref,
                    gam_ref, bet_ref,
                    w1a_ref, w1b_ref, eb1_ref,
                    nw1_ref, nb1_ref, nw2_ref, nb2_ref,
                    h_out, a_out, b_out, nl_out):
    h = h_ref[...]
    agg = g0_ref[...] + g1_ref[...]
    t = _dot(h, v1h_ref[...]) + _dot(agg, v1a_ref[...]) + vb1_ref[...]
    y = h + _dot(_gelu(t), v2_ref[...]) + vb2_ref[...]
    hn = _ln(y, gam_ref[...], bet_ref[...])
    h_out[...] = hn
    a_out[...] = _dot(hn, w1a_ref[...]) + eb1_ref[...]
    b_out[...] = _dot(hn, w1b_ref[...])
    u = _gelu(_dot(hn, nw1_ref[...]) + nb1_ref[...])
    nl_out[...] = _dot(u, nw2_ref[...]) + nb2_ref[...]


_node_last = pl.pallas_call(
    _node_last_body,
    grid=(GN,),
    in_specs=[
        pl.BlockSpec((TN, H), lambda i: (i, 0)),
        pl.BlockSpec((TN, H), lambda i: (i, 0)),
        pl.BlockSpec((TN, H), lambda i: (i, 0)),
        _mat_spec(), _mat_spec(), _vec_spec(), _mat_spec(), _vec_spec(),
        _vec_spec(), _vec_spec(),
        _mat_spec(), _mat_spec(), _vec_spec(),
        _mat_spec(), _vec_spec(),
        pl.BlockSpec((H, 1), lambda i: (0, 0)),
        pl.BlockSpec((1, 1), lambda i: (0, 0)),
    ],
    out_specs=[
        pl.BlockSpec((TN, H), lambda i: (i, 0)),
        pl.BlockSpec((TN, H), lambda i: (i, 0)),
        pl.BlockSpec((TN, H), lambda i: (i, 0)),
        pl.BlockSpec((TN, 1), lambda i: (i, 0)),
    ],
    out_shape=[
        jax.ShapeDtypeStruct((N, H), jnp.float32),
        jax.ShapeDtypeStruct((N, H), jnp.float32),
        jax.ShapeDtypeStruct((N, H), jnp.float32),
        jax.ShapeDtypeStruct((N, 1), jnp.float32),
    ],
)


def _edge_head_body(e_ref, g_ref, w1c_ref, w2_ref, b2_ref, out_ref):
    t = g_ref[...] + _dot(e_ref[...], w1c_ref[...])
    out_ref[...] = _dot(_gelu(t), w2_ref[...]) + b2_ref[...]


_edge_head = pl.pallas_call(
    _edge_head_body,
    grid=(GE,),
    in_specs=[
        pl.BlockSpec((TE, H), lambda i: (i, 0)),
        pl.BlockSpec((TE, H), lambda i: (i, 0)),
        _mat_spec(),
        pl.BlockSpec((H, 1), lambda i: (0, 0)),
        pl.BlockSpec((1, 1), lambda i: (0, 0)),
    ],
    out_specs=pl.BlockSpec((TE, 1), lambda i: (i, 0)),
    out_shape=jax.ShapeDtypeStruct((E, 1), jnp.float32),
)


def _row(v):
    return v.reshape(1, -1)


def kernel(x, edge_index, edge_attr, params):
    src = edge_index[0]
    dst = edge_index[1]

    blocks = params["blocks"]
    ew = [bp["edge_mlp"]["l1"]["w"] for bp in blocks]
    ew.append(params["edge_head"]["l1"]["w"])
    eb = [bp["edge_mlp"]["l1"]["b"] for bp in blocks]
    eb.append(params["edge_head"]["l1"]["b"])
    w1a = [w[:H] for w in ew]
    w1b = [w[H:2 * H] for w in ew]
    w1c = [w[2 * H:] for w in ew]

    xp = jnp.pad(x, ((0, 0), (0, 2)))
    ne = params["node_enc"]
    nw1 = jnp.pad(ne["l1"]["w"], ((0, 2), (0, 0)))
    nh = params["node_head"]

    h, a, b = _node_enc(xp, nw1, _row(ne["l1"]["b"]),
                        ne["l2"]["w"], _row(ne["l2"]["b"]),
                        w1a[0], w1b[0], _row(eb[0]))

    ee = params["edge_enc"]
    ew1 = ee["l1"]["w"]
    e = _edge_enc(edge_attr, ew1, _row(ee["l1"]["b"]),
                  ee["l2"]["w"], _row(ee["l2"]["b"]))

    for k, bp in enumerate(blocks):
        g = _gather_pair(a, b, src, dst)
        e = _edge_update(e, g, w1c[k],
                         bp["edge_mlp"]["l2"]["w"],
                         _row(bp["edge_mlp"]["l2"]["b"]),
                         _row(bp["ln_e"]["g"]), _row(bp["ln_e"]["b"]))
        agg = _scatter_sum(e, dst)
        nm = bp["node_mlp"]
        v1 = nm["l1"]["w"]
        common = (h, agg[0], agg[1],
                  v1[:H], v1[H:], _row(nm["l1"]["b"]),
                  nm["l2"]["w"], _row(nm["l2"]["b"]),
                  _row(bp["ln_h"]["g"]), _row(bp["ln_h"]["b"]),
                  w1a[k + 1], w1b[k + 1], _row(eb[k + 1]))
        if k + 1 < len(blocks):
            h, a, b = _node_update(*common)
        else:
            h, a, b, nl = _node_last(
                *common,
                nh["l1"]["w"], _row(nh["l1"]["b"]),
                nh["l2"]["w"], _row(nh["l2"]["b"]))

    g = _gather_pair(a, b, src, dst)
    hd = params["edge_head"]
    el = _edge_head(e, g, w1c[4],
                    hd["l2"]["w"], _row(hd["l2"]["b"]))

    return (el[:, 0], nl[:, 0])
